# baseline (speedup 1.0000x reference)
import functools

import jax
import jax.numpy as jnp
from jax import lax
from jax.experimental import pallas as pl
from jax.experimental.pallas import tpu as pltpu
from jax.experimental.pallas import tpu_sc as plsc

_NC = 2
_NS = 16
_NW = _NC * _NS

_CHUNK = 2048
_L = 16


def _lookup_call(B, H, D, Vp):
    n = (B * H) // _CHUNK
    cols = B // _CHUNK
    mesh = plsc.VectorSubcoreMesh(core_axis_name="c", subcore_axis_name="s")

    @functools.partial(
        pl.kernel,
        mesh=mesh,
        out_type=jax.ShapeDtypeStruct((H, D, B), jnp.float32),
        scratch_types=[
            pltpu.VMEM((Vp,), jnp.float32),
            pltpu.VMEM((_CHUNK,), jnp.int32),
            pltpu.VMEM((_CHUNK,), jnp.int32),
            pltpu.VMEM((_CHUNK,), jnp.float32),
            pltpu.VMEM((_CHUNK,), jnp.float32),
            pltpu.SemaphoreType.DMA,
            pltpu.SemaphoreType.DMA,
            pltpu.SemaphoreType.DMA,
            pltpu.SemaphoreType.DMA,
        ],
        compiler_params=pltpu.CompilerParams(
            use_tc_tiling_on_sc=False, needs_layout_passes=False),
    )
    def run(tab_hbm, idx_hbm, out_hbm, plane, idx0, idx1, res0, res1,
            isem0, isem1, ssem0, ssem1):
        wid = lax.axis_index("s") * _NC + lax.axis_index("c")

        pltpu.sync_copy(tab_hbm.at[wid], plane)

        def idx_cp(g, idx_b, isem_b):
            h = g // cols
            c = g % cols
            return pltpu.make_async_copy(
                idx_hbm.at[h, pl.ds(c * _CHUNK, _CHUNK)], idx_b, isem_b)

        def store_cp(g, res_b, ssem_b):
            h = g // cols
            c = g % cols
            return pltpu.make_async_copy(
                res_b, out_hbm.at[h, wid, pl.ds(c * _CHUNK, _CHUNK)], ssem_b)

        def gather_chunk(idx_b, res_b):
            def inner(k, carry):
                iv = idx_b[pl.ds(k * _L, _L)]
                res_b[pl.ds(k * _L, _L)] = plsc.load_gather(plane, [iv])
                return carry
            lax.fori_loop(0, _CHUNK // _L, inner, 0, unroll=8)

        idx_cp(0, idx0, isem0).start()
        idx_cp(0, idx0, isem0).wait()
        idx_cp(1, idx1, isem1).start()
        gather_chunk(idx0, res0)
        store_cp(0, res0, ssem0).start()

        def body(t, carry):
            g0 = 2 * t + 1
            idx_cp(g0, idx1, isem1).wait()
            idx_cp(g0 + 1, idx0, isem0).start()

            @pl.when(t > 0)
            def _():
                store_cp(g0 - 2, res1, ssem1).wait()

            gather_chunk(idx1, res1)
            store_cp(g0, res1, ssem1).start()
            idx_cp(g0 + 1, idx0, isem0).wait()

            @pl.when(g0 + 2 < n)
            def _():
                idx_cp(g0 + 2, idx1, isem1).start()

            store_cp(g0 - 1, res0, ssem0).wait()
            gather_chunk(idx0, res0)
            store_cp(g0 + 1, res0, ssem0).start()
            return carry

        lax.fori_loop(0, n // 2 - 1, body, 0)

        idx_cp(n - 1, idx1, isem1).wait()
        store_cp(n - 3, res1, ssem1).wait()
        gather_chunk(idx1, res1)
        store_cp(n - 1, res1, ssem1).start()
        store_cp(n - 2, res0, ssem0).wait()
        store_cp(n - 1, res1, ssem1).wait()

    return run


def kernel(X, table):
    B, H = X.shape
    V, D = table.shape
    Vp = ((V + 127) // 128) * 128
    idx = X.T
    tab = jnp.pad(table.T, ((0, 0), (0, Vp - V)))
    out_t = _lookup_call(B, H, D, Vp)(tab, idx)
    return out_t.transpose(2, 0, 1)


# speedup vs baseline: 1.3820x; 1.3820x over previous
import functools

import jax
import jax.numpy as jnp
from jax import lax
from jax.experimental import pallas as pl
from jax.experimental.pallas import tpu as pltpu
from jax.experimental.pallas import tpu_sc as plsc

_NC = 2
_NS = 16
_NW = _NC * _NS

_CHUNK = 2048
_L = 16


def _lookup_call(B, H, D, Vp):
    n = (B * H) // _CHUNK
    cols = B // _CHUNK
    mesh = plsc.VectorSubcoreMesh(core_axis_name="c", subcore_axis_name="s")

    @functools.partial(
        pl.kernel,
        mesh=mesh,
        out_type=jax.ShapeDtypeStruct((H, D, B), jnp.float32),
        scratch_types=[
            pltpu.VMEM((Vp,), jnp.float32),
            pltpu.VMEM((_CHUNK,), jnp.int32),
            pltpu.VMEM((_CHUNK,), jnp.int32),
            pltpu.VMEM((_CHUNK,), jnp.float32),
            pltpu.VMEM((_CHUNK,), jnp.float32),
            pltpu.SemaphoreType.DMA,
            pltpu.SemaphoreType.DMA,
            pltpu.SemaphoreType.DMA,
            pltpu.SemaphoreType.DMA,
        ],
        compiler_params=pltpu.CompilerParams(
            use_tc_tiling_on_sc=False, needs_layout_passes=False),
    )
    def run(tab_hbm, idx_hbm, out_hbm, plane, idx0, idx1, res0, res1,
            isem0, isem1, ssem0, ssem1):
        wid = lax.axis_index("s") * _NC + lax.axis_index("c")

        pltpu.sync_copy(tab_hbm.at[wid], plane)

        def idx_cp(g, idx_b, isem_b):
            h = g // cols
            c = g % cols
            return pltpu.make_async_copy(
                idx_hbm.at[h, pl.ds(c * _CHUNK, _CHUNK)], idx_b, isem_b)

        def store_cp(g, res_b, ssem_b):
            h = g // cols
            c = g % cols
            return pltpu.make_async_copy(
                res_b, out_hbm.at[h, wid, pl.ds(c * _CHUNK, _CHUNK)], ssem_b)

        def gather_chunk(idx_b, res_b):
            for k in range(_CHUNK // _L):
                iv = idx_b[pl.ds(k * _L, _L)]
                res_b[pl.ds(k * _L, _L)] = plsc.load_gather(plane, [iv])

        idx_cp(0, idx0, isem0).start()
        idx_cp(0, idx0, isem0).wait()
        idx_cp(1, idx1, isem1).start()
        gather_chunk(idx0, res0)
        store_cp(0, res0, ssem0).start()

        def body(t, carry):
            g0 = 2 * t + 1
            idx_cp(g0, idx1, isem1).wait()
            idx_cp(g0 + 1, idx0, isem0).start()

            @pl.when(t > 0)
            def _():
                store_cp(g0 - 2, res1, ssem1).wait()

            gather_chunk(idx1, res1)
            store_cp(g0, res1, ssem1).start()
            idx_cp(g0 + 1, idx0, isem0).wait()

            @pl.when(g0 + 2 < n)
            def _():
                idx_cp(g0 + 2, idx1, isem1).start()

            store_cp(g0 - 1, res0, ssem0).wait()
            gather_chunk(idx0, res0)
            store_cp(g0 + 1, res0, ssem0).start()
            return carry

        lax.fori_loop(0, n // 2 - 1, body, 0)

        idx_cp(n - 1, idx1, isem1).wait()
        store_cp(n - 3, res1, ssem1).wait()
        gather_chunk(idx1, res1)
        store_cp(n - 1, res1, ssem1).start()
        store_cp(n - 2, res0, ssem0).wait()
        store_cp(n - 1, res1, ssem1).wait()

    return run


def kernel(X, table):
    B, H = X.shape
    V, D = table.shape
    Vp = ((V + 127) // 128) * 128
    idx = X.T
    tab = jnp.pad(table.T, ((0, 0), (0, Vp - V)))
    out_t = _lookup_call(B, H, D, Vp)(tab, idx)
    return out_t.transpose(2, 0, 1)


# speedup vs baseline: 1.3856x; 1.0026x over previous
import functools

import jax
import jax.numpy as jnp
from jax import lax
from jax.experimental import pallas as pl
from jax.experimental.pallas import tpu as pltpu
from jax.experimental.pallas import tpu_sc as plsc

_NC = 2
_NS = 16
_NW = _NC * _NS

_CHUNK = 2048
_L = 16


def _lookup_call(B, H, D, Vp):
    n = (B * H) // _CHUNK
    cols = B // _CHUNK
    mesh = plsc.VectorSubcoreMesh(core_axis_name="c", subcore_axis_name="s")

    @functools.partial(
        pl.kernel,
        mesh=mesh,
        out_type=jax.ShapeDtypeStruct((H, D, B), jnp.float32),
        scratch_types=[
            pltpu.VMEM((Vp,), jnp.float32),
            pltpu.VMEM((_CHUNK,), jnp.int32),
            pltpu.VMEM((_CHUNK,), jnp.int32),
            pltpu.VMEM((_CHUNK,), jnp.float32),
            pltpu.VMEM((_CHUNK,), jnp.float32),
            pltpu.SemaphoreType.DMA,
            pltpu.SemaphoreType.DMA,
            pltpu.SemaphoreType.DMA,
            pltpu.SemaphoreType.DMA,
        ],
        compiler_params=pltpu.CompilerParams(
            use_tc_tiling_on_sc=False, needs_layout_passes=False),
    )
    def run(tab_hbm, idx_hbm, out_hbm, plane, idx0, idx1, res0, res1,
            isem0, isem1, ssem0, ssem1):
        wid = lax.axis_index("s") * _NC + lax.axis_index("c")

        pltpu.sync_copy(tab_hbm.at[wid], plane)

        def idx_cp(g, idx_b, isem_b):
            h = g // cols
            c = g % cols
            return pltpu.make_async_copy(
                idx_hbm.at[h, pl.ds(c * _CHUNK, _CHUNK)], idx_b, isem_b)

        def store_cp(g, res_b, ssem_b):
            h = g // cols
            c = g % cols
            return pltpu.make_async_copy(
                res_b, out_hbm.at[h, wid, pl.ds(c * _CHUNK, _CHUNK)], ssem_b)

        def gather_chunk(idx_b, res_b):
            @plsc.parallel_loop(0, _CHUNK, step=_L, unroll=8)
            def _(i):
                iv = idx_b[pl.ds(i, _L)]
                res_b[pl.ds(i, _L)] = plsc.load_gather(plane, [iv])

        idx_cp(0, idx0, isem0).start()
        idx_cp(0, idx0, isem0).wait()
        idx_cp(1, idx1, isem1).start()
        gather_chunk(idx0, res0)
        store_cp(0, res0, ssem0).start()

        def body(t, carry):
            g0 = 2 * t + 1
            idx_cp(g0, idx1, isem1).wait()
            idx_cp(g0 + 1, idx0, isem0).start()

            @pl.when(t > 0)
            def _():
                store_cp(g0 - 2, res1, ssem1).wait()

            gather_chunk(idx1, res1)
            store_cp(g0, res1, ssem1).start()
            idx_cp(g0 + 1, idx0, isem0).wait()

            @pl.when(g0 + 2 < n)
            def _():
                idx_cp(g0 + 2, idx1, isem1).start()

            store_cp(g0 - 1, res0, ssem0).wait()
            gather_chunk(idx0, res0)
            store_cp(g0 + 1, res0, ssem0).start()
            return carry

        lax.fori_loop(0, n // 2 - 1, body, 0)

        idx_cp(n - 1, idx1, isem1).wait()
        store_cp(n - 3, res1, ssem1).wait()
        gather_chunk(idx1, res1)
        store_cp(n - 1, res1, ssem1).start()
        store_cp(n - 2, res0, ssem0).wait()
        store_cp(n - 1, res1, ssem1).wait()

    return run


def kernel(X, table):
    B, H = X.shape
    V, D = table.shape
    Vp = ((V + 127) // 128) * 128
    idx = X.T
    tab = jnp.pad(table.T, ((0, 0), (0, Vp - V)))
    out_t = _lookup_call(B, H, D, Vp)(tab, idx)
    return out_t.transpose(2, 0, 1)


# speedup vs baseline: 1.3873x; 1.0012x over previous
import functools

import jax
import jax.numpy as jnp
from jax import lax
from jax.experimental import pallas as pl
from jax.experimental.pallas import tpu as pltpu
from jax.experimental.pallas import tpu_sc as plsc

_NC = 2
_NS = 16
_NW = _NC * _NS

_CHUNK = 2048
_L = 16


def _lookup_call(B, H, D, Vp):
    n = (B * H) // _CHUNK
    cols = B // _CHUNK
    mesh = plsc.VectorSubcoreMesh(core_axis_name="c", subcore_axis_name="s")

    @functools.partial(
        pl.kernel,
        mesh=mesh,
        out_type=jax.ShapeDtypeStruct((H, D, B), jnp.float32),
        scratch_types=[
            pltpu.VMEM((Vp,), jnp.float32),
            pltpu.VMEM((_CHUNK,), jnp.int32),
            pltpu.VMEM((_CHUNK,), jnp.int32),
            pltpu.VMEM((_CHUNK,), jnp.float32),
            pltpu.VMEM((_CHUNK,), jnp.float32),
            pltpu.SemaphoreType.DMA,
            pltpu.SemaphoreType.DMA,
            pltpu.SemaphoreType.DMA,
            pltpu.SemaphoreType.DMA,
        ],
        compiler_params=pltpu.CompilerParams(
            use_tc_tiling_on_sc=False, needs_layout_passes=False),
    )
    def run(tab_hbm, idx_hbm, out_hbm, plane, idx0, idx1, res0, res1,
            isem0, isem1, ssem0, ssem1):
        wid = lax.axis_index("s") * _NC + lax.axis_index("c")

        pltpu.sync_copy(tab_hbm.at[wid], plane)

        def idx_cp(g, idx_b, isem_b):
            h = g // cols
            c = g % cols
            return pltpu.make_async_copy(
                idx_hbm.at[h, pl.ds(c * _CHUNK, _CHUNK)], idx_b, isem_b)

        def store_cp(g, res_b, ssem_b):
            h = g // cols
            c = g % cols
            return pltpu.make_async_copy(
                res_b, out_hbm.at[h, wid, pl.ds(c * _CHUNK, _CHUNK)], ssem_b)

        def gather_chunk(idx_b, res_b):
            w = 4
            for k in range(_CHUNK // (_L * w)):
                ivs = [idx_b[pl.ds((k * w + j) * _L, _L)] for j in range(w)]
                rvs = [plsc.load_gather(plane, [iv]) for iv in ivs]
                for j in range(w):
                    res_b[pl.ds((k * w + j) * _L, _L)] = rvs[j]

        idx_cp(0, idx0, isem0).start()
        idx_cp(0, idx0, isem0).wait()
        idx_cp(1, idx1, isem1).start()
        gather_chunk(idx0, res0)
        store_cp(0, res0, ssem0).start()

        def body(t, carry):
            g0 = 2 * t + 1
            idx_cp(g0, idx1, isem1).wait()
            idx_cp(g0 + 1, idx0, isem0).start()

            @pl.when(t > 0)
            def _():
                store_cp(g0 - 2, res1, ssem1).wait()

            gather_chunk(idx1, res1)
            store_cp(g0, res1, ssem1).start()
            idx_cp(g0 + 1, idx0, isem0).wait()

            @pl.when(g0 + 2 < n)
            def _():
                idx_cp(g0 + 2, idx1, isem1).start()

            store_cp(g0 - 1, res0, ssem0).wait()
            gather_chunk(idx0, res0)
            store_cp(g0 + 1, res0, ssem0).start()
            return carry

        lax.fori_loop(0, n // 2 - 1, body, 0)

        idx_cp(n - 1, idx1, isem1).wait()
        store_cp(n - 3, res1, ssem1).wait()
        gather_chunk(idx1, res1)
        store_cp(n - 1, res1, ssem1).start()
        store_cp(n - 2, res0, ssem0).wait()
        store_cp(n - 1, res1, ssem1).wait()

    return run


def kernel(X, table):
    B, H = X.shape
    V, D = table.shape
    Vp = ((V + 127) // 128) * 128
    idx = X.T
    tab = jnp.pad(table.T, ((0, 0), (0, Vp - V)))
    out_t = _lookup_call(B, H, D, Vp)(tab, idx)
    return out_t.transpose(2, 0, 1)


# speedup vs baseline: 1.8518x; 1.3348x over previous
import functools

import jax
import jax.numpy as jnp
from jax import lax
from jax.experimental import pallas as pl
from jax.experimental.pallas import tpu as pltpu
from jax.experimental.pallas import tpu_sc as plsc

_NC = 2
_NS = 16
_NW = _NC * _NS

_CHUNK = 4096
_L = 16


def _lookup_call(B, H, D, Vp):
    n = (B * H) // _CHUNK
    cols = B // _CHUNK
    mesh = plsc.VectorSubcoreMesh(core_axis_name="c", subcore_axis_name="s")

    @functools.partial(
        pl.kernel,
        mesh=mesh,
        out_type=jax.ShapeDtypeStruct((H, D, B), jnp.float32),
        scratch_types=[
            pltpu.VMEM((Vp,), jnp.float32),
            pltpu.VMEM((_CHUNK,), jnp.int32),
            pltpu.VMEM((_CHUNK,), jnp.int32),
            pltpu.VMEM((_CHUNK,), jnp.float32),
            pltpu.VMEM((_CHUNK,), jnp.float32),
            pltpu.SemaphoreType.DMA,
            pltpu.SemaphoreType.DMA,
            pltpu.SemaphoreType.DMA,
            pltpu.SemaphoreType.DMA,
        ],
        compiler_params=pltpu.CompilerParams(
            use_tc_tiling_on_sc=False, needs_layout_passes=False),
    )
    def run(tab_hbm, idx_hbm, out_hbm, plane, idx0, idx1, res0, res1,
            isem0, isem1, ssem0, ssem1):
        wid = lax.axis_index("s") * _NC + lax.axis_index("c")

        pltpu.sync_copy(tab_hbm.at[wid], plane)

        def idx_cp(g, idx_b, isem_b):
            h = g // cols
            c = g % cols
            return pltpu.make_async_copy(
                idx_hbm.at[h, pl.ds(c * _CHUNK, _CHUNK)], idx_b, isem_b)

        def store_cp(g, res_b, ssem_b):
            h = g // cols
            c = g % cols
            return pltpu.make_async_copy(
                res_b, out_hbm.at[h, wid, pl.ds(c * _CHUNK, _CHUNK)], ssem_b)

        def gather_chunk(idx_b, res_b):
            w = 4
            for k in range(_CHUNK // (_L * w)):
                ivs = [idx_b[pl.ds((k * w + j) * _L, _L)] for j in range(w)]
                rvs = [plsc.load_gather(plane, [iv]) for iv in ivs]
                for j in range(w):
                    res_b[pl.ds((k * w + j) * _L, _L)] = rvs[j]

        idx_cp(0, idx0, isem0).start()
        idx_cp(0, idx0, isem0).wait()
        idx_cp(1, idx1, isem1).start()
        gather_chunk(idx0, res0)
        store_cp(0, res0, ssem0).start()

        def body(t, carry):
            g0 = 2 * t + 1
            idx_cp(g0, idx1, isem1).wait()
            idx_cp(g0 + 1, idx0, isem0).start()

            @pl.when(t > 0)
            def _():
                store_cp(g0 - 2, res1, ssem1).wait()

            gather_chunk(idx1, res1)
            store_cp(g0, res1, ssem1).start()
            idx_cp(g0 + 1, idx0, isem0).wait()

            @pl.when(g0 + 2 < n)
            def _():
                idx_cp(g0 + 2, idx1, isem1).start()

            store_cp(g0 - 1, res0, ssem0).wait()
            gather_chunk(idx0, res0)
            store_cp(g0 + 1, res0, ssem0).start()
            return carry

        lax.fori_loop(0, n // 2 - 1, body, 0)

        idx_cp(n - 1, idx1, isem1).wait()
        store_cp(n - 3, res1, ssem1).wait()
        gather_chunk(idx1, res1)
        store_cp(n - 1, res1, ssem1).start()
        store_cp(n - 2, res0, ssem0).wait()
        store_cp(n - 1, res1, ssem1).wait()

    return run


def kernel(X, table):
    B, H = X.shape
    V, D = table.shape
    Vp = ((V + 127) // 128) * 128
    idx = X.T
    tab = jnp.pad(table.T, ((0, 0), (0, Vp - V)))
    out_t = _lookup_call(B, H, D, Vp)(tab, idx)
    return out_t.transpose(2, 0, 1)
